# baseline (speedup 1.0000x reference)
import functools

import jax
import jax.numpy as jnp
from jax import lax
from jax.experimental import pallas as pl
from jax.experimental.pallas import tpu as pltpu
from jax.experimental.pallas import tpu_sc as plsc

VOCAB = 1000
EMBED_DIM = 64
BATCH = 1024
HIST = 200

NUM_CORES = 2
NUM_SUBCORES = 16
NW = NUM_CORES * NUM_SUBCORES
NBT = BATCH // 128
H_PER_W = HIST // (NW // NBT)
LANES = 16
NGRP = 128 // LANES

_mesh = plsc.VectorSubcoreMesh(core_axis_name="c", subcore_axis_name="s")


@functools.partial(
    pl.kernel,
    out_type=jax.ShapeDtypeStruct((HIST, 8, NBT, 8, 128), jnp.float32),
    mesh=_mesh,
    scratch_types=[
        pltpu.VMEM((VOCAB * EMBED_DIM,), jnp.float32),
        pltpu.VMEM((H_PER_W, 128), jnp.int32),
        pltpu.VMEM((2, 8, 8, 128), jnp.float32),
        [pltpu.SemaphoreType.DMA] * 2,
    ],
    compiler_params=pltpu.CompilerParams(
        use_tc_tiling_on_sc=False, needs_layout_passes=False
    ),
)
def _lookup(idx_hbm, table_hbm, out_hbm, table_v, idx_v, block_v, wsems):
    wid = lax.axis_index("s") * NUM_CORES + lax.axis_index("c")
    bt = wid % NBT
    h0 = (wid // NBT) * H_PER_W

    pltpu.sync_copy(table_hbm, table_v)
    pltpu.sync_copy(
        idx_hbm.at[pl.ds(h0, H_PER_W), pl.ds(bt * 128, 128)], idx_v
    )

    iota = lax.iota(jnp.int32, LANES)

    def compute(u, buf):
        def grp(g, carry):
            tok = idx_v[u, pl.ds(g * LANES, LANES)]
            base = tok * EMBED_DIM
            bvec = iota + g * LANES

            def win(chi, c2):
                base2 = base + chi * 16
                row0 = 2 * chi
                for j0 in range(0, 16, 16):
                    ds = [(iota + (j0 + k)) & 15 for k in range(16)]
                    vals = [
                        plsc.load_gather(table_v, [base2 + ds[k]])
                        for k in range(16)
                    ]
                    for k in range(16):
                        plsc.store_scatter(
                            block_v.at[buf],
                            [(ds[k] >> 3) + row0, ds[k] & 7, bvec],
                            vals[k],
                        )
                return c2

            lax.fori_loop(0, EMBED_DIM // 16, win, 0)
            return carry

        lax.fori_loop(0, NGRP, grp, 0)

    def start_w(u, buf):
        pltpu.async_copy(
            block_v.at[buf], out_hbm.at[h0 + u, :, bt], wsems[buf]
        )

    def wait_w(u, buf):
        pltpu.make_async_copy(
            block_v.at[buf], out_hbm.at[h0 + u, :, bt], wsems[buf]
        ).wait()

    compute(0, 0)
    start_w(0, 0)
    compute(1, 1)
    start_w(1, 1)

    def outer(i, carry):
        for b2 in range(2):
            u = 2 * i + b2
            wait_w(u - 2, b2)
            compute(u, b2)
            start_w(u, b2)
        return carry

    lax.fori_loop(1, H_PER_W // 2, outer, 0)
    wait_w(H_PER_W - 2, 0)
    wait_w(H_PER_W - 1, 1)


def kernel(token_indices, embeddings):
    out5 = _lookup(token_indices.T, embeddings.reshape(VOCAB * EMBED_DIM))
    return out5.transpose(2, 4, 0, 1, 3).reshape(BATCH, HIST, EMBED_DIM)


# speedup vs baseline: 1.1562x; 1.1562x over previous
import functools

import jax
import jax.numpy as jnp
from jax import lax
from jax.experimental import pallas as pl
from jax.experimental.pallas import tpu as pltpu
from jax.experimental.pallas import tpu_sc as plsc

VOCAB = 1000
EMBED_DIM = 64
BATCH = 1024
HIST = 200

NUM_CORES = 2
NUM_SUBCORES = 16
NW = NUM_CORES * NUM_SUBCORES
NBT = BATCH // 128
H_PER_W = HIST // (NW // NBT)
LANES = 16
NGRP = 128 // LANES

_mesh = plsc.VectorSubcoreMesh(core_axis_name="c", subcore_axis_name="s")


@functools.partial(
    pl.kernel,
    out_type=jax.ShapeDtypeStruct((HIST, 8, NBT, 8, 128), jnp.float32),
    mesh=_mesh,
    scratch_types=[
        pltpu.VMEM((VOCAB * EMBED_DIM,), jnp.float32),
        pltpu.VMEM((H_PER_W, 128), jnp.int32),
        pltpu.VMEM((2, 8, 8, 128), jnp.float32),
        [pltpu.SemaphoreType.DMA] * 2,
    ],
    compiler_params=pltpu.CompilerParams(
        use_tc_tiling_on_sc=False, needs_layout_passes=False
    ),
)
def _lookup(idx_hbm, table_hbm, out_hbm, table_v, idx_v, block_v, wsems):
    wid = lax.axis_index("s") * NUM_CORES + lax.axis_index("c")
    bt = wid % NBT
    h0 = (wid // NBT) * H_PER_W

    pltpu.sync_copy(table_hbm, table_v)
    pltpu.sync_copy(
        idx_hbm.at[pl.ds(h0, H_PER_W), pl.ds(bt * 128, 128)], idx_v
    )

    iota = lax.iota(jnp.int32, LANES)

    def compute(u, buf):
        @plsc.parallel_loop(0, NGRP)
        def grp(g):
            tok = idx_v[u, pl.ds(g * LANES, LANES)]
            base = tok * EMBED_DIM
            bvec = iota + g * LANES

            @plsc.parallel_loop(0, EMBED_DIM // 16)
            def win(chi):
                base2 = base + chi * 16
                row0 = 2 * chi
                for j0 in range(0, 16, 8):
                    ds = [(iota + (j0 + k)) & 15 for k in range(8)]
                    vals = [
                        plsc.load_gather(table_v, [base2 + ds[k]])
                        for k in range(8)
                    ]
                    for k in range(8):
                        plsc.store_scatter(
                            block_v.at[buf],
                            [(ds[k] >> 3) + row0, ds[k] & 7, bvec],
                            vals[k],
                        )

    def start_w(u, buf):
        pltpu.async_copy(
            block_v.at[buf], out_hbm.at[h0 + u, :, bt], wsems[buf]
        )

    def wait_w(u, buf):
        pltpu.make_async_copy(
            block_v.at[buf], out_hbm.at[h0 + u, :, bt], wsems[buf]
        ).wait()

    compute(0, 0)
    start_w(0, 0)
    compute(1, 1)
    start_w(1, 1)

    def outer(i, carry):
        for b2 in range(2):
            u = 2 * i + b2
            wait_w(u - 2, b2)
            compute(u, b2)
            start_w(u, b2)
        return carry

    lax.fori_loop(1, H_PER_W // 2, outer, 0)
    wait_w(H_PER_W - 2, 0)
    wait_w(H_PER_W - 1, 1)


def kernel(token_indices, embeddings):
    out5 = _lookup(token_indices.T, embeddings.reshape(VOCAB * EMBED_DIM))
    return out5.transpose(2, 4, 0, 1, 3).reshape(BATCH, HIST, EMBED_DIM)
